# baseline (speedup 1.0000x reference)
import functools

import jax
import jax.numpy as jnp
import numpy as np
from jax import lax
from jax.experimental import pallas as pl
from jax.experimental.pallas import tpu as pltpu
from jax.experimental.pallas import tpu_sc as plsc

N_LEVELS = 16
F = 2
LOG2_T = 19
T = 1 << LOG2_T
MASK = T - 1
BASE_RES = 16
PER_LEVEL_SCALE = float(np.exp2(np.log2(2048.0 * 1 * 1 / 16.0) / (16 - 1)))
N_POINTS = 524288

RES = [int(np.floor(BASE_RES * (PER_LEVEL_SCALE ** l))) for l in range(N_LEVELS)]
DENSE = [(r + 1) ** 3 <= T for r in RES]

P1 = int(np.uint32(2654435761).astype(np.int32))
P2 = int(np.uint32(805459861).astype(np.int32))

NW = 32
N_CACHED = 2
CACHE_ROWS0 = (RES[0] + 1) ** 3
CACHE_ROWS1 = (RES[1] + 1) ** 3
CACHE_OFF1 = ((CACHE_ROWS0 * 2 + 15) // 16) * 16
CACHE_LEN1 = ((CACHE_ROWS1 * 2 + 15) // 16) * 16
CACHE_SIZE = CACHE_OFF1 + CACHE_LEN1
C = 512
PTS_PER_W = N_POINTS // NW
NCHUNK = PTS_PER_W // C
NG = C // 16


def _positions(xref, s, res):
    resf = jnp.float32(float(res))
    px = xref[pl.ds(s, 16)] * resf
    py = xref[pl.ds(C + s, 16)] * resf
    pz = xref[pl.ds(2 * C + s, 16)] * resf
    return px, py, pz


def _build_idx(l, half, xref, iref):
    res = RES[l]
    base = jnp.int32(l * T)
    off = half * 8 * C

    def body(g, carry):
        s = g * 16
        px, py, pz = _positions(xref, s, res)
        ix = px.astype(jnp.int32)
        iy = py.astype(jnp.int32)
        iz = pz.astype(jnp.int32)
        if DENSE[l]:
            stride = jnp.int32(res + 1)
            stride2 = jnp.int32((res + 1) * (res + 1))
            tx = (ix + base, ix + (base + 1))
            ty0 = iy * stride
            ty = (ty0, ty0 + stride)
            tz0 = iz * stride2
            tz = (tz0, tz0 + stride2)
            for k in range(8):
                bx, by, bz = k & 1, (k >> 1) & 1, (k >> 2) & 1
                iref[pl.ds(off + k * C + s, 16)] = tx[bx] + ty[by] + tz[bz]
        else:
            hx = (ix, ix + 1)
            hy0 = iy * P1
            hy = (hy0, hy0 + P1)
            hz0 = iz * P2
            hz = (hz0, hz0 + P2)
            for k in range(8):
                bx, by, bz = k & 1, (k >> 1) & 1, (k >> 2) & 1
                h = (hx[bx] ^ hy[by] ^ hz[bz]) & MASK
                iref[pl.ds(off + k * C + s, 16)] = h + base
        return carry

    lax.fori_loop(0, NG, body, 0, unroll=False)


def _accumulate(l, half, xref, rref, oref):
    res = RES[l]
    roff = half * 8 * C
    zeros = jnp.zeros((16,), jnp.int32)
    ones = jnp.ones((16,), jnp.int32)
    lane = lax.iota(jnp.int32, 16)

    def body(g, carry):
        s = g * 16
        px, py, pz = _positions(xref, s, res)
        ix = px.astype(jnp.int32)
        iy = py.astype(jnp.int32)
        iz = pz.astype(jnp.int32)
        wx1 = px - ix.astype(jnp.float32)
        wy1 = py - iy.astype(jnp.float32)
        wz1 = pz - iz.astype(jnp.float32)
        wx0 = 1.0 - wx1
        wy0 = 1.0 - wy1
        wz0 = 1.0 - wz1
        wxy = (wx0 * wy0, wx1 * wy0, wx0 * wy1, wx1 * wy1)
        wz = (wz0, wz1)
        acc0 = jnp.zeros((16,), jnp.float32)
        acc1 = jnp.zeros((16,), jnp.float32)
        for k in range(8):
            bx, by, bz = k & 1, (k >> 1) & 1, (k >> 2) & 1
            wc = wxy[by * 2 + bx] * wz[bz]
            row = (roff + k * C + s) + lane
            f0 = plsc.load_gather(rref, [row, zeros])
            f1 = plsc.load_gather(rref, [row, ones])
            acc0 = acc0 + wc * f0
            acc1 = acc1 + wc * f1
        oref[pl.ds(2 * l * C + s, 16)] = acc0
        oref[pl.ds((2 * l + 1) * C + s, 16)] = acc1
        return carry

    lax.fori_loop(0, NG, body, 0, unroll=False)


def _fused_cached(l, row_off, xref, cref, oref):
    res = RES[l]
    stride = res + 1

    def body(g, carry):
        s = g * 16
        px, py, pz = _positions(xref, s, res)
        ix = px.astype(jnp.int32)
        iy = py.astype(jnp.int32)
        iz = pz.astype(jnp.int32)
        wx1 = px - ix.astype(jnp.float32)
        wy1 = py - iy.astype(jnp.float32)
        wz1 = pz - iz.astype(jnp.float32)
        wxy = (
            (1.0 - wx1) * (1.0 - wy1), wx1 * (1.0 - wy1),
            (1.0 - wx1) * wy1, wx1 * wy1,
        )
        wz = (1.0 - wz1, wz1)
        tx = (ix * 2 + (2 * row_off), ix * 2 + (2 * row_off + 2))
        ty0 = iy * (2 * stride)
        ty = (ty0, ty0 + 2 * stride)
        tz0 = iz * (2 * stride * stride)
        tz = (tz0, tz0 + 2 * stride * stride)
        acc0 = jnp.zeros((16,), jnp.float32)
        acc1 = jnp.zeros((16,), jnp.float32)
        for k in range(8):
            bx, by, bz = k & 1, (k >> 1) & 1, (k >> 2) & 1
            wc = wxy[by * 2 + bx] * wz[bz]
            e0 = tx[bx] + ty[by] + tz[bz]
            f0 = plsc.load_gather(cref, [e0 >> 4, e0 & 15])
            f1 = plsc.load_gather(cref, [(e0 + 1) >> 4, (e0 + 1) & 15])
            acc0 = acc0 + wc * f0
            acc1 = acc1 + wc * f1
        oref[2 * l, pl.ds(s, 16)] = acc0
        oref[2 * l + 1, pl.ds(s, 16)] = acc1
        return carry

    lax.fori_loop(0, NG, body, 0, unroll=False)


def _encode_body(xt_hbm, tab_hbm, enc_hbm, xbuf, ibufs, rbufs, obuf, sems):
    wid = lax.axis_index("s") * 2 + lax.axis_index("c")

    def chunk_body(ci, carry):
        base = (wid * NCHUNK + ci) * C
        for j in range(3):
            pltpu.sync_copy(xt_hbm.at[pl.ds(j * N_POINTS + base, C)],
                            xbuf.at[pl.ds(j * C, C)])

        _build_idx(0, 0, xbuf, ibufs[0])
        copies = [None, None]
        copies[0] = pltpu.async_copy(tab_hbm.at[ibufs[0]], rbufs[0], sems[0])
        for l in range(1, N_LEVELS):
            a, b = l % 2, (l - 1) % 2
            _build_idx(l, 0, xbuf, ibufs[a])
            copies[a] = pltpu.async_copy(tab_hbm.at[ibufs[a]], rbufs[a],
                                         sems[a])
            copies[b].wait()
            _accumulate(l - 1, 0, xbuf, rbufs[b], obuf)
        last = (N_LEVELS - 1) % 2
        copies[last].wait()
        _accumulate(N_LEVELS - 1, 0, xbuf, rbufs[last], obuf)

        blk = wid * NCHUNK + ci
        pltpu.sync_copy(obuf, enc_hbm.at[pl.ds(blk * (2 * N_LEVELS * C),
                                               2 * N_LEVELS * C)])
        return carry

    lax.fori_loop(0, NCHUNK, chunk_body, 0, unroll=False)


def _encode(xt, tab):
    mesh = plsc.VectorSubcoreMesh(core_axis_name="c", subcore_axis_name="s")
    kfn = pl.kernel(
        lambda xt_hbm, tab_hbm, enc_hbm, xbuf, i0, i1, r0, r1, obuf, \
               s0, s1: (
            _encode_body(xt_hbm, tab_hbm, enc_hbm, xbuf, (i0, i1), (r0, r1),
                         obuf, (s0, s1))
        ),
        out_type=jax.ShapeDtypeStruct((2 * N_LEVELS * N_POINTS,),
                                      jnp.float32),
        mesh=mesh,
        compiler_params=pltpu.CompilerParams(needs_layout_passes=False,
                                             use_tc_tiling_on_sc=False),
        scratch_types=[
            pltpu.VMEM((3 * C,), jnp.float32),
            pltpu.VMEM((8 * C,), jnp.int32),
            pltpu.VMEM((8 * C,), jnp.int32),
            pltpu.VMEM((8 * C, F), jnp.float32),
            pltpu.VMEM((8 * C, F), jnp.float32),
            pltpu.VMEM((2 * N_LEVELS * C,), jnp.float32),
            pltpu.SemaphoreType.DMA,
            pltpu.SemaphoreType.DMA,
        ],
    )
    return kfn(xt, tab)


def _transpose_body(x_ref, o0_ref, o1_ref, o2_ref):
    blk = x_ref[...]
    o0_ref[...] = blk[:, 0]
    o1_ref[...] = blk[:, 1]
    o2_ref[...] = blk[:, 2]


def _transpose_x(x):
    bn = 16384
    outs = pl.pallas_call(
        _transpose_body,
        grid=(N_POINTS // bn,),
        in_specs=[pl.BlockSpec((bn, 3), lambda i: (i, 0))],
        out_specs=[pl.BlockSpec((bn,), lambda i: (i,))] * 3,
        out_shape=[jax.ShapeDtypeStruct((N_POINTS,), jnp.float32)] * 3,
    )(x)
    return jnp.concatenate(outs)


def _softplus_b10(v):
    z = 10.0 * v
    return (jnp.maximum(z, 0.0) + jnp.log1p(jnp.exp(-jnp.abs(z)))) * 0.1


def _mlp_body(e_ref, w0_ref, w1_ref, w2_ref, w3_ref, o_ref):
    dn = (((1,), (0,)), ((), ()))
    e = e_ref[...].reshape(2 * N_LEVELS, C)
    h = lax.dot_general(w0_ref[...], e, dn,
                        preferred_element_type=jnp.float32)
    h = _softplus_b10(h)
    h = lax.dot_general(w1_ref[...], h, dn, preferred_element_type=jnp.float32)
    h = _softplus_b10(h)
    h = lax.dot_general(w2_ref[...], h, dn, preferred_element_type=jnp.float32)
    h = _softplus_b10(h)
    o_ref[...] = lax.dot_general(w3_ref[...], h, dn,
                                 preferred_element_type=jnp.float32)


def _mlp(enc_t, W0, W1, W2, W3):
    grid = (N_POINTS // C,)
    return pl.pallas_call(
        _mlp_body,
        grid=grid,
        in_specs=[
            pl.BlockSpec((2 * N_LEVELS * C,), lambda i: (i,)),
            pl.BlockSpec((64, 32), lambda i: (0, 0)),
            pl.BlockSpec((64, 64), lambda i: (0, 0)),
            pl.BlockSpec((64, 64), lambda i: (0, 0)),
            pl.BlockSpec((1, 64), lambda i: (0, 0)),
        ],
        out_specs=pl.BlockSpec((1, C), lambda i: (0, i)),
        out_shape=jax.ShapeDtypeStruct((1, N_POINTS), jnp.float32),
    )(enc_t, W0, W1, W2, W3)


@jax.jit
def kernel(x, table, W0, W1, W2, W3):
    xt = _transpose_x(x)
    tab = table.reshape(N_LEVELS * T, F)
    enc_t = _encode(xt, tab)
    sdf = _mlp(enc_t, W0, W1, W2, W3)
    return sdf.reshape(N_POINTS, 1)


# speedup vs baseline: 2.6413x; 2.6413x over previous
import functools

import jax
import jax.numpy as jnp
import numpy as np
from jax import lax
from jax.experimental import pallas as pl
from jax.experimental.pallas import tpu as pltpu
from jax.experimental.pallas import tpu_sc as plsc

N_LEVELS = 16
F = 2
LOG2_T = 19
T = 1 << LOG2_T
MASK = T - 1
BASE_RES = 16
PER_LEVEL_SCALE = float(np.exp2(np.log2(2048.0 * 1 * 1 / 16.0) / (16 - 1)))
N_POINTS = 524288

RES = [int(np.floor(BASE_RES * (PER_LEVEL_SCALE ** l))) for l in range(N_LEVELS)]
DENSE = [(r + 1) ** 3 <= T for r in RES]

P1 = int(np.uint32(2654435761).astype(np.int32))
P2 = int(np.uint32(805459861).astype(np.int32))

NW = 32
N_CACHED = 2
CACHE_ROWS0 = (RES[0] + 1) ** 3
CACHE_ROWS1 = (RES[1] + 1) ** 3
CACHE_OFF1 = ((CACHE_ROWS0 * 2 + 15) // 16) * 16
CACHE_LEN1 = ((CACHE_ROWS1 * 2 + 15) // 16) * 16
CACHE_SIZE = CACHE_OFF1 + CACHE_LEN1
C = 512
PTS_PER_W = N_POINTS // NW
NCHUNK = PTS_PER_W // C
NG = C // 16


def _positions(xref, s, res):
    resf = jnp.float32(float(res))
    px = xref[pl.ds(s, 16)] * resf
    py = xref[pl.ds(C + s, 16)] * resf
    pz = xref[pl.ds(2 * C + s, 16)] * resf
    return px, py, pz


def _build_idx(l, half, xref, iref):
    res = RES[l]
    lbase = jnp.int32(l << 20)

    def body(g, carry):
        s = g * 16
        px, py, pz = _positions(xref, s, res)
        ix = px.astype(jnp.int32)
        iy = py.astype(jnp.int32)
        iz = pz.astype(jnp.int32)
        if DENSE[l]:
            stride = jnp.int32(res + 1)
            stride2 = jnp.int32((res + 1) * (res + 1))
            tx = (ix, ix + 1)
            ty0 = iy * stride
            ty = (ty0, ty0 + stride)
            tz0 = iz * stride2
            tz = (tz0, tz0 + stride2)
        else:
            hy0 = iy * P1
            hy = (hy0, hy0 + P1)
            hz0 = iz * P2
            hz = (hz0, hz0 + P2)
        for k in range(8):
            bx, by, bz = k & 1, (k >> 1) & 1, (k >> 2) & 1
            if DENSE[l]:
                t = tx[bx] + ty[by] + tz[bz]
            else:
                t = ((ix + bx) ^ hy[by] ^ hz[bz]) & MASK
            e0 = lbase + ((t >> 7) << 8) + (t & 127)
            iref[pl.ds(k * C + s, 16)] = e0
            iref[pl.ds(8 * C + k * C + s, 16)] = e0 + 128
        return carry

    lax.fori_loop(0, NG, body, 0, unroll=False)


def _accumulate(l, half, xref, rref, oref):
    res = RES[l]

    def body(g, carry):
        s = g * 16
        px, py, pz = _positions(xref, s, res)
        ix = px.astype(jnp.int32)
        iy = py.astype(jnp.int32)
        iz = pz.astype(jnp.int32)
        wx1 = px - ix.astype(jnp.float32)
        wy1 = py - iy.astype(jnp.float32)
        wz1 = pz - iz.astype(jnp.float32)
        wx0 = 1.0 - wx1
        wy0 = 1.0 - wy1
        wz0 = 1.0 - wz1
        wxy = (wx0 * wy0, wx1 * wy0, wx0 * wy1, wx1 * wy1)
        wz = (wz0, wz1)
        acc0 = jnp.zeros((16,), jnp.float32)
        acc1 = jnp.zeros((16,), jnp.float32)
        for k in range(8):
            bx, by, bz = k & 1, (k >> 1) & 1, (k >> 2) & 1
            wc = wxy[by * 2 + bx] * wz[bz]
            f0 = rref[pl.ds(k * C + s, 16)]
            f1 = rref[pl.ds(8 * C + k * C + s, 16)]
            acc0 = acc0 + wc * f0
            acc1 = acc1 + wc * f1
        oref[pl.ds(2 * l * C + s, 16)] = acc0
        oref[pl.ds((2 * l + 1) * C + s, 16)] = acc1
        return carry

    lax.fori_loop(0, NG, body, 0, unroll=False)


def _fused_cached(l, row_off, xref, cref, oref):
    res = RES[l]
    stride = res + 1

    def body(g, carry):
        s = g * 16
        px, py, pz = _positions(xref, s, res)
        ix = px.astype(jnp.int32)
        iy = py.astype(jnp.int32)
        iz = pz.astype(jnp.int32)
        wx1 = px - ix.astype(jnp.float32)
        wy1 = py - iy.astype(jnp.float32)
        wz1 = pz - iz.astype(jnp.float32)
        wxy = (
            (1.0 - wx1) * (1.0 - wy1), wx1 * (1.0 - wy1),
            (1.0 - wx1) * wy1, wx1 * wy1,
        )
        wz = (1.0 - wz1, wz1)
        tx = (ix * 2 + (2 * row_off), ix * 2 + (2 * row_off + 2))
        ty0 = iy * (2 * stride)
        ty = (ty0, ty0 + 2 * stride)
        tz0 = iz * (2 * stride * stride)
        tz = (tz0, tz0 + 2 * stride * stride)
        acc0 = jnp.zeros((16,), jnp.float32)
        acc1 = jnp.zeros((16,), jnp.float32)
        for k in range(8):
            bx, by, bz = k & 1, (k >> 1) & 1, (k >> 2) & 1
            wc = wxy[by * 2 + bx] * wz[bz]
            e0 = tx[bx] + ty[by] + tz[bz]
            f0 = plsc.load_gather(cref, [e0 >> 4, e0 & 15])
            f1 = plsc.load_gather(cref, [(e0 + 1) >> 4, (e0 + 1) & 15])
            acc0 = acc0 + wc * f0
            acc1 = acc1 + wc * f1
        oref[2 * l, pl.ds(s, 16)] = acc0
        oref[2 * l + 1, pl.ds(s, 16)] = acc1
        return carry

    lax.fori_loop(0, NG, body, 0, unroll=False)


def _encode_body(xt_hbm, tab_hbm, enc_hbm, xbuf, ibufs, rbufs, obuf, sems):
    wid = lax.axis_index("s") * 2 + lax.axis_index("c")

    def chunk_body(ci, carry):
        base = (wid * NCHUNK + ci) * C
        for j in range(3):
            pltpu.sync_copy(xt_hbm.at[pl.ds(j * N_POINTS + base, C)],
                            xbuf.at[pl.ds(j * C, C)])

        _build_idx(0, 0, xbuf, ibufs[0])
        copies = [None, None]
        copies[0] = pltpu.async_copy(tab_hbm.at[ibufs[0]], rbufs[0], sems[0])
        for l in range(1, N_LEVELS):
            a, b = l % 2, (l - 1) % 2
            _build_idx(l, 0, xbuf, ibufs[a])
            copies[a] = pltpu.async_copy(tab_hbm.at[ibufs[a]], rbufs[a],
                                         sems[a])
            copies[b].wait()
            _accumulate(l - 1, 0, xbuf, rbufs[b], obuf)
        last = (N_LEVELS - 1) % 2
        copies[last].wait()
        _accumulate(N_LEVELS - 1, 0, xbuf, rbufs[last], obuf)

        blk = wid * NCHUNK + ci
        pltpu.sync_copy(obuf, enc_hbm.at[pl.ds(blk * (2 * N_LEVELS * C),
                                               2 * N_LEVELS * C)])
        return carry

    lax.fori_loop(0, NCHUNK, chunk_body, 0, unroll=False)


def _encode(xt, tab):
    mesh = plsc.VectorSubcoreMesh(core_axis_name="c", subcore_axis_name="s")
    kfn = pl.kernel(
        lambda xt_hbm, tab_hbm, enc_hbm, xbuf, i0, i1, r0, r1, obuf, \
               s0, s1: (
            _encode_body(xt_hbm, tab_hbm, enc_hbm, xbuf, (i0, i1), (r0, r1),
                         obuf, (s0, s1))
        ),
        out_type=jax.ShapeDtypeStruct((2 * N_LEVELS * N_POINTS,),
                                      jnp.float32),
        mesh=mesh,
        compiler_params=pltpu.CompilerParams(needs_layout_passes=False,
                                             use_tc_tiling_on_sc=False),
        scratch_types=[
            pltpu.VMEM((3 * C,), jnp.float32),
            pltpu.VMEM((16 * C,), jnp.int32),
            pltpu.VMEM((16 * C,), jnp.int32),
            pltpu.VMEM((16 * C,), jnp.float32),
            pltpu.VMEM((16 * C,), jnp.float32),
            pltpu.VMEM((2 * N_LEVELS * C,), jnp.float32),
            pltpu.SemaphoreType.DMA,
            pltpu.SemaphoreType.DMA,
        ],
    )
    return kfn(xt, tab)


def _transpose_body(x_ref, o0_ref, o1_ref, o2_ref):
    blk = x_ref[...]
    o0_ref[...] = blk[:, 0]
    o1_ref[...] = blk[:, 1]
    o2_ref[...] = blk[:, 2]


def _transpose_x(x):
    bn = 16384
    outs = pl.pallas_call(
        _transpose_body,
        grid=(N_POINTS // bn,),
        in_specs=[pl.BlockSpec((bn, 3), lambda i: (i, 0))],
        out_specs=[pl.BlockSpec((bn,), lambda i: (i,))] * 3,
        out_shape=[jax.ShapeDtypeStruct((N_POINTS,), jnp.float32)] * 3,
    )(x)
    return jnp.concatenate(outs)


def _softplus_b10(v):
    z = 10.0 * v
    return (jnp.maximum(z, 0.0) + jnp.log1p(jnp.exp(-jnp.abs(z)))) * 0.1


def _mlp_body(e_ref, w0_ref, w1_ref, w2_ref, w3_ref, o_ref):
    dn = (((1,), (0,)), ((), ()))
    e = e_ref[...].reshape(2 * N_LEVELS, C)
    h = lax.dot_general(w0_ref[...], e, dn,
                        preferred_element_type=jnp.float32)
    h = _softplus_b10(h)
    h = lax.dot_general(w1_ref[...], h, dn, preferred_element_type=jnp.float32)
    h = _softplus_b10(h)
    h = lax.dot_general(w2_ref[...], h, dn, preferred_element_type=jnp.float32)
    h = _softplus_b10(h)
    o_ref[...] = lax.dot_general(w3_ref[...], h, dn,
                                 preferred_element_type=jnp.float32)


def _mlp(enc_t, W0, W1, W2, W3):
    grid = (N_POINTS // C,)
    return pl.pallas_call(
        _mlp_body,
        grid=grid,
        in_specs=[
            pl.BlockSpec((2 * N_LEVELS * C,), lambda i: (i,)),
            pl.BlockSpec((64, 32), lambda i: (0, 0)),
            pl.BlockSpec((64, 64), lambda i: (0, 0)),
            pl.BlockSpec((64, 64), lambda i: (0, 0)),
            pl.BlockSpec((1, 64), lambda i: (0, 0)),
        ],
        out_specs=pl.BlockSpec((1, C), lambda i: (0, i)),
        out_shape=jax.ShapeDtypeStruct((1, N_POINTS), jnp.float32),
    )(enc_t, W0, W1, W2, W3)


@jax.jit
def kernel(x, table, W0, W1, W2, W3):
    xt = _transpose_x(x)
    tab = table.reshape(N_LEVELS, T // 128, 128, F).transpose(
        0, 1, 3, 2).reshape(N_LEVELS * T * F)
    enc_t = _encode(xt, tab)
    sdf = _mlp(enc_t, W0, W1, W2, W3)
    return sdf.reshape(N_POINTS, 1)


# speedup vs baseline: 2.7626x; 1.0459x over previous
import functools

import jax
import jax.numpy as jnp
import numpy as np
from jax import lax
from jax.experimental import pallas as pl
from jax.experimental.pallas import tpu as pltpu
from jax.experimental.pallas import tpu_sc as plsc

N_LEVELS = 16
F = 2
LOG2_T = 19
T = 1 << LOG2_T
MASK = T - 1
BASE_RES = 16
PER_LEVEL_SCALE = float(np.exp2(np.log2(2048.0 * 1 * 1 / 16.0) / (16 - 1)))
N_POINTS = 524288

RES = [int(np.floor(BASE_RES * (PER_LEVEL_SCALE ** l))) for l in range(N_LEVELS)]
DENSE = [(r + 1) ** 3 <= T for r in RES]

P1 = int(np.uint32(2654435761).astype(np.int32))
P2 = int(np.uint32(805459861).astype(np.int32))

NW = 32
N_CACHED = 2
CACHE_ROWS0 = (RES[0] + 1) ** 3
CACHE_ROWS1 = (RES[1] + 1) ** 3
CACHE_OFF1 = ((CACHE_ROWS0 * 2 + 15) // 16) * 16
CACHE_LEN1 = ((CACHE_ROWS1 * 2 + 15) // 16) * 16
CACHE_SIZE = CACHE_OFF1 + CACHE_LEN1
C = 1024
PTS_PER_W = N_POINTS // NW
NCHUNK = PTS_PER_W // C
NG = C // 16


def _positions(xref, s, res):
    resf = jnp.float32(float(res))
    px = xref[pl.ds(s, 16)] * resf
    py = xref[pl.ds(C + s, 16)] * resf
    pz = xref[pl.ds(2 * C + s, 16)] * resf
    return px, py, pz


def _build_idx(l, half, xref, iref):
    res = RES[l]
    lbase = jnp.int32(l << 20)

    def body(g, carry):
        s = g * 16
        px, py, pz = _positions(xref, s, res)
        ix = px.astype(jnp.int32)
        iy = py.astype(jnp.int32)
        iz = pz.astype(jnp.int32)
        if DENSE[l]:
            stride = jnp.int32(res + 1)
            stride2 = jnp.int32((res + 1) * (res + 1))
            tx = (ix, ix + 1)
            ty0 = iy * stride
            ty = (ty0, ty0 + stride)
            tz0 = iz * stride2
            tz = (tz0, tz0 + stride2)
        else:
            hy0 = iy * P1
            hy = (hy0, hy0 + P1)
            hz0 = iz * P2
            hz = (hz0, hz0 + P2)
        for k in range(8):
            bx, by, bz = k & 1, (k >> 1) & 1, (k >> 2) & 1
            if DENSE[l]:
                t = tx[bx] + ty[by] + tz[bz]
            else:
                t = ((ix + bx) ^ hy[by] ^ hz[bz]) & MASK
            e0 = lbase + ((t >> 7) << 8) + (t & 127)
            iref[pl.ds(k * C + s, 16)] = e0
            iref[pl.ds(8 * C + k * C + s, 16)] = e0 + 128
        return carry

    lax.fori_loop(0, NG, body, 0, unroll=False)


def _accumulate(l, half, xref, rref, oref):
    res = RES[l]

    def body(g, carry):
        s = g * 16
        px, py, pz = _positions(xref, s, res)
        ix = px.astype(jnp.int32)
        iy = py.astype(jnp.int32)
        iz = pz.astype(jnp.int32)
        wx1 = px - ix.astype(jnp.float32)
        wy1 = py - iy.astype(jnp.float32)
        wz1 = pz - iz.astype(jnp.float32)
        wx0 = 1.0 - wx1
        wy0 = 1.0 - wy1
        wz0 = 1.0 - wz1
        wxy = (wx0 * wy0, wx1 * wy0, wx0 * wy1, wx1 * wy1)
        wz = (wz0, wz1)
        acc0 = jnp.zeros((16,), jnp.float32)
        acc1 = jnp.zeros((16,), jnp.float32)
        for k in range(8):
            bx, by, bz = k & 1, (k >> 1) & 1, (k >> 2) & 1
            wc = wxy[by * 2 + bx] * wz[bz]
            f0 = rref[pl.ds(k * C + s, 16)]
            f1 = rref[pl.ds(8 * C + k * C + s, 16)]
            acc0 = acc0 + wc * f0
            acc1 = acc1 + wc * f1
        oref[pl.ds(2 * l * C + s, 16)] = acc0
        oref[pl.ds((2 * l + 1) * C + s, 16)] = acc1
        return carry

    lax.fori_loop(0, NG, body, 0, unroll=False)


def _fused_cached(l, row_off, xref, cref, oref):
    res = RES[l]
    stride = res + 1

    def body(g, carry):
        s = g * 16
        px, py, pz = _positions(xref, s, res)
        ix = px.astype(jnp.int32)
        iy = py.astype(jnp.int32)
        iz = pz.astype(jnp.int32)
        wx1 = px - ix.astype(jnp.float32)
        wy1 = py - iy.astype(jnp.float32)
        wz1 = pz - iz.astype(jnp.float32)
        wxy = (
            (1.0 - wx1) * (1.0 - wy1), wx1 * (1.0 - wy1),
            (1.0 - wx1) * wy1, wx1 * wy1,
        )
        wz = (1.0 - wz1, wz1)
        tx = (ix * 2 + (2 * row_off), ix * 2 + (2 * row_off + 2))
        ty0 = iy * (2 * stride)
        ty = (ty0, ty0 + 2 * stride)
        tz0 = iz * (2 * stride * stride)
        tz = (tz0, tz0 + 2 * stride * stride)
        acc0 = jnp.zeros((16,), jnp.float32)
        acc1 = jnp.zeros((16,), jnp.float32)
        for k in range(8):
            bx, by, bz = k & 1, (k >> 1) & 1, (k >> 2) & 1
            wc = wxy[by * 2 + bx] * wz[bz]
            e0 = tx[bx] + ty[by] + tz[bz]
            f0 = plsc.load_gather(cref, [e0 >> 4, e0 & 15])
            f1 = plsc.load_gather(cref, [(e0 + 1) >> 4, (e0 + 1) & 15])
            acc0 = acc0 + wc * f0
            acc1 = acc1 + wc * f1
        oref[2 * l, pl.ds(s, 16)] = acc0
        oref[2 * l + 1, pl.ds(s, 16)] = acc1
        return carry

    lax.fori_loop(0, NG, body, 0, unroll=False)


def _encode_body(xt_hbm, tab_hbm, enc_hbm, xbuf, ibufs, rbufs, obuf, sems):
    wid = lax.axis_index("s") * 2 + lax.axis_index("c")

    def chunk_body(ci, carry):
        base = (wid * NCHUNK + ci) * C
        for j in range(3):
            pltpu.sync_copy(xt_hbm.at[pl.ds(j * N_POINTS + base, C)],
                            xbuf.at[pl.ds(j * C, C)])

        _build_idx(0, 0, xbuf, ibufs[0])
        copies = [None, None]
        copies[0] = pltpu.async_copy(tab_hbm.at[ibufs[0]], rbufs[0], sems[0])
        for l in range(1, N_LEVELS):
            a, b = l % 2, (l - 1) % 2
            _build_idx(l, 0, xbuf, ibufs[a])
            copies[a] = pltpu.async_copy(tab_hbm.at[ibufs[a]], rbufs[a],
                                         sems[a])
            copies[b].wait()
            _accumulate(l - 1, 0, xbuf, rbufs[b], obuf)
        last = (N_LEVELS - 1) % 2
        copies[last].wait()
        _accumulate(N_LEVELS - 1, 0, xbuf, rbufs[last], obuf)

        blk = wid * NCHUNK + ci
        pltpu.sync_copy(obuf, enc_hbm.at[pl.ds(blk * (2 * N_LEVELS * C),
                                               2 * N_LEVELS * C)])
        return carry

    lax.fori_loop(0, NCHUNK, chunk_body, 0, unroll=False)


def _encode(xt, tab):
    mesh = plsc.VectorSubcoreMesh(core_axis_name="c", subcore_axis_name="s")
    kfn = pl.kernel(
        lambda xt_hbm, tab_hbm, enc_hbm, xbuf, i0, i1, r0, r1, obuf, \
               s0, s1: (
            _encode_body(xt_hbm, tab_hbm, enc_hbm, xbuf, (i0, i1), (r0, r1),
                         obuf, (s0, s1))
        ),
        out_type=jax.ShapeDtypeStruct((2 * N_LEVELS * N_POINTS,),
                                      jnp.float32),
        mesh=mesh,
        compiler_params=pltpu.CompilerParams(needs_layout_passes=False,
                                             use_tc_tiling_on_sc=False),
        scratch_types=[
            pltpu.VMEM((3 * C,), jnp.float32),
            pltpu.VMEM((16 * C,), jnp.int32),
            pltpu.VMEM((16 * C,), jnp.int32),
            pltpu.VMEM((16 * C,), jnp.float32),
            pltpu.VMEM((16 * C,), jnp.float32),
            pltpu.VMEM((2 * N_LEVELS * C,), jnp.float32),
            pltpu.SemaphoreType.DMA,
            pltpu.SemaphoreType.DMA,
        ],
    )
    return kfn(xt, tab)


def _transpose_body(x_ref, o0_ref, o1_ref, o2_ref):
    blk = x_ref[...]
    o0_ref[...] = blk[:, 0]
    o1_ref[...] = blk[:, 1]
    o2_ref[...] = blk[:, 2]


def _transpose_x(x):
    bn = 16384
    outs = pl.pallas_call(
        _transpose_body,
        grid=(N_POINTS // bn,),
        in_specs=[pl.BlockSpec((bn, 3), lambda i: (i, 0))],
        out_specs=[pl.BlockSpec((bn,), lambda i: (i,))] * 3,
        out_shape=[jax.ShapeDtypeStruct((N_POINTS,), jnp.float32)] * 3,
    )(x)
    return jnp.concatenate(outs)


def _softplus_b10(v):
    z = 10.0 * v
    return (jnp.maximum(z, 0.0) + jnp.log1p(jnp.exp(-jnp.abs(z)))) * 0.1


def _mlp_body(e_ref, w0_ref, w1_ref, w2_ref, w3_ref, o_ref):
    dn = (((1,), (0,)), ((), ()))
    e = e_ref[...].reshape(2 * N_LEVELS, C)
    h = lax.dot_general(w0_ref[...], e, dn,
                        preferred_element_type=jnp.float32)
    h = _softplus_b10(h)
    h = lax.dot_general(w1_ref[...], h, dn, preferred_element_type=jnp.float32)
    h = _softplus_b10(h)
    h = lax.dot_general(w2_ref[...], h, dn, preferred_element_type=jnp.float32)
    h = _softplus_b10(h)
    o_ref[...] = lax.dot_general(w3_ref[...], h, dn,
                                 preferred_element_type=jnp.float32)


def _mlp(enc_t, W0, W1, W2, W3):
    grid = (N_POINTS // C,)
    return pl.pallas_call(
        _mlp_body,
        grid=grid,
        in_specs=[
            pl.BlockSpec((2 * N_LEVELS * C,), lambda i: (i,)),
            pl.BlockSpec((64, 32), lambda i: (0, 0)),
            pl.BlockSpec((64, 64), lambda i: (0, 0)),
            pl.BlockSpec((64, 64), lambda i: (0, 0)),
            pl.BlockSpec((1, 64), lambda i: (0, 0)),
        ],
        out_specs=pl.BlockSpec((1, C), lambda i: (0, i)),
        out_shape=jax.ShapeDtypeStruct((1, N_POINTS), jnp.float32),
    )(enc_t, W0, W1, W2, W3)


@jax.jit
def kernel(x, table, W0, W1, W2, W3):
    xt = _transpose_x(x)
    tab = table.reshape(N_LEVELS, T // 128, 128, F).transpose(
        0, 1, 3, 2).reshape(N_LEVELS * T * F)
    enc_t = _encode(xt, tab)
    sdf = _mlp(enc_t, W0, W1, W2, W3)
    return sdf.reshape(N_POINTS, 1)
